# baseline (speedup 1.0000x reference)
import jax
import jax.numpy as jnp
from jax import lax
from jax.experimental import pallas as pl
from jax.experimental.pallas import tpu as pltpu
from jax.experimental.pallas import tpu_sc as plsc

_B = 4
_S = 8192
_H = 768
_EPS = 1e-12
_NC = 2
_NS = 16
_NW = _NC * _NS
_SPW = _S // _NW
_P = 16
_KR = _P * _B
_NROUND = _SPW // _P
_L = 16


def _nr_rsqrt(v):
    i = lax.bitcast_convert_type(v, jnp.int32)
    y = lax.bitcast_convert_type(
        jnp.int32(0x5F3759DF) - lax.shift_right_arithmetic(i, 1), jnp.float32)
    for _ in range(2):
        y = y * (1.5 - 0.5 * v * y * y)
    return y


def _tree_sum(vs):
    vs = list(vs)
    while len(vs) > 1:
        nxt = [vs[i] + vs[i + 1] for i in range(0, len(vs) - 1, 2)]
        if len(vs) % 2:
            nxt.append(vs[-1])
        vs = nxt
    return vs[0]


def _body(ids_hbm, tok_hbm, pos_hbm, gamma_hbm, beta_hbm, out_hbm,
          idx_all, pos_v, tok_a, tok_b, gam_v, bet_v, st_v,
          gsem_a, gsem_b, osem_a, osem_b):
    cid = lax.axis_index("c")
    sid = lax.axis_index("s")
    wid = sid * _NC + cid
    s_base = wid * _SPW

    pltpu.sync_copy(gamma_hbm, gam_v)
    pltpu.sync_copy(beta_hbm, bet_v)
    for b in range(_B):
        pltpu.sync_copy(ids_hbm.at[b, pl.ds(s_base, _SPW)],
                        idx_all.at[pl.ds(b * _SPW, _SPW)])

    inv = jnp.float32(1.0 / _H)
    nchunk = _H // _L

    def gathers(r, tok_buf, gsem):
        for b in range(_B):
            pltpu.async_copy(
                tok_hbm.at[idx_all.at[pl.ds(b * _SPW + r * _P, _P)]],
                tok_buf.at[pl.ds(b * _P, _P)], gsem)

    def gwait(r, tok_buf, gsem):
        for b in range(_B):
            pltpu.make_async_copy(
                tok_hbm.at[idx_all.at[pl.ds(b * _SPW + r * _P, _P)]],
                tok_buf.at[pl.ds(b * _P, _P)], gsem).wait()

    def outs(r, tok_buf, osem):
        for b in range(_B):
            pltpu.async_copy(tok_buf.at[pl.ds(b * _P, _P)],
                             out_hbm.at[b, pl.ds(s_base + r * _P, _P)], osem)

    def owait(r, tok_buf, osem):
        for b in range(_B):
            pltpu.make_async_copy(
                tok_buf.at[pl.ds(b * _P, _P)],
                out_hbm.at[b, pl.ds(s_base + r * _P, _P)], osem).wait()

    def compute_chunk(tok_buf):
        def p1(i):
            vs = [[] for _ in range(_B)]
            for cc in range(nchunk):
                sl = pl.ds(cc * _L, _L)
                pv = pos_v[i, sl]
                for b in range(_B):
                    v = tok_buf[b * _P + i, sl] + pv
                    tok_buf[b * _P + i, sl] = v
                    vs[b].append(v)
            for b in range(_B):
                s1 = _tree_sum(vs[b])
                s2 = _tree_sum([v * v for v in vs[b]])
                mv = jnp.full((_L,), jnp.sum(s1)) * inv
                qv = jnp.full((_L,), jnp.sum(s2)) * inv
                rr = _nr_rsqrt(qv - mv * mv + jnp.float32(_EPS))
                st_v[b * _P + i, pl.ds(0, _L)] = rr
                st_v[b * _P + i, pl.ds(_L, _L)] = -(mv * rr)

        plsc.parallel_loop(0, _P, 1)(p1)

        strip = 16
        for s in range(nchunk // strip):
            gs = [gam_v[pl.ds((s * strip + j) * _L, _L)] for j in range(strip)]
            bs = [bet_v[pl.ds((s * strip + j) * _L, _L)] for j in range(strip)]

            def p2(rw, _gs=gs, _bs=bs, _s=s):
                rv = st_v[rw, pl.ds(0, _L)]
                nv = st_v[rw, pl.ds(_L, _L)]
                for j in range(strip):
                    sl = pl.ds((_s * strip + j) * _L, _L)
                    x = tok_buf[rw, sl]
                    tok_buf[rw, sl] = (x * rv + nv) * _gs[j] + _bs[j]

            plsc.parallel_loop(0, _KR, 1, unroll=2)(p2)

    def do_round(r, tok_cur, gsem_cur, osem_cur, tok_nxt, gsem_nxt, osem_nxt):
        pltpu.sync_copy(pos_hbm.at[pl.ds(s_base + r * _P, _P)], pos_v)

        gwait(r, tok_cur, gsem_cur)

        @pl.when(r < _NROUND - 1)
        def _():
            @pl.when(r >= 1)
            def _():
                owait(r, tok_nxt, osem_nxt)
            gathers(r + 1, tok_nxt, gsem_nxt)

        compute_chunk(tok_cur)
        outs(r, tok_cur, osem_cur)

    gathers(jnp.int32(0), tok_a, gsem_a)

    def pair(jj, _):
        r0 = jj * 2
        do_round(r0, tok_a, gsem_a, osem_a, tok_b, gsem_b, osem_b)
        do_round(r0 + 1, tok_b, gsem_b, osem_b, tok_a, gsem_a, osem_a)
        return 0

    lax.fori_loop(0, _NROUND // 2, pair, 0)

    owait(jnp.int32(_NROUND - 2), tok_a, osem_a)
    owait(jnp.int32(_NROUND - 1), tok_b, osem_b)


_mesh = plsc.VectorSubcoreMesh(
    core_axis_name="c", subcore_axis_name="s", num_cores=_NC, num_subcores=_NS)

_embed_ln = pl.kernel(
    _body,
    out_type=jax.ShapeDtypeStruct((_B, _S, _H), jnp.float32),
    mesh=_mesh,
    scratch_types=[
        pltpu.VMEM((_B * _SPW,), jnp.int32),
        pltpu.VMEM((_P, _H), jnp.float32),
        pltpu.VMEM((_KR, _H), jnp.float32),
        pltpu.VMEM((_KR, _H), jnp.float32),
        pltpu.VMEM((_H,), jnp.float32),
        pltpu.VMEM((_H,), jnp.float32),
        pltpu.VMEM((_KR, 2 * _L), jnp.float32),
        pltpu.SemaphoreType.DMA,
        pltpu.SemaphoreType.DMA,
        pltpu.SemaphoreType.DMA,
        pltpu.SemaphoreType.DMA,
    ],
    compiler_params=pltpu.CompilerParams(
        use_tc_tiling_on_sc=True, needs_layout_passes=False),
)


def kernel(input_ids, tok_table, pos_table, gamma, beta):
    return _embed_ln(input_ids.astype(jnp.int32), tok_table, pos_table,
                     gamma, beta)


# speedup vs baseline: 1.0666x; 1.0666x over previous
import jax
import jax.numpy as jnp
from jax import lax
from jax.experimental import pallas as pl
from jax.experimental.pallas import tpu as pltpu
from jax.experimental.pallas import tpu_sc as plsc

_B = 4
_S = 8192
_H = 768
_EPS = 1e-12
_NC = 2
_NS = 16
_NW = _NC * _NS
_SPW = _S // _NW
_P = 16
_KR = _P * _B
_NROUND = _SPW // _P
_L = 16


def _nr_rsqrt(v):
    i = lax.bitcast_convert_type(v, jnp.int32)
    y = lax.bitcast_convert_type(
        jnp.int32(0x5F3759DF) - lax.shift_right_arithmetic(i, 1), jnp.float32)
    for _ in range(2):
        y = y * (1.5 - 0.5 * v * y * y)
    return y


def _tree_sum(vs):
    vs = list(vs)
    while len(vs) > 1:
        nxt = [vs[i] + vs[i + 1] for i in range(0, len(vs) - 1, 2)]
        if len(vs) % 2:
            nxt.append(vs[-1])
        vs = nxt
    return vs[0]


def _body(ids_hbm, tok_hbm, pos_hbm, gamma_hbm, beta_hbm, out_hbm,
          idx_all, pos_v, tok_a, tok_b, gam_v, bet_v, st_v,
          gsem_a, gsem_b, osem_a, osem_b):
    cid = lax.axis_index("c")
    sid = lax.axis_index("s")
    wid = sid * _NC + cid
    s_base = wid * _SPW

    pltpu.sync_copy(gamma_hbm, gam_v)
    pltpu.sync_copy(beta_hbm, bet_v)
    for b in range(_B):
        pltpu.sync_copy(ids_hbm.at[b, pl.ds(s_base, _SPW)],
                        idx_all.at[pl.ds(b * _SPW, _SPW)])

    inv = jnp.float32(1.0 / _H)
    nchunk = _H // _L

    def gathers(r, tok_buf, gsem):
        for b in range(_B):
            pltpu.async_copy(
                tok_hbm.at[idx_all.at[pl.ds(b * _SPW + r * _P, _P)]],
                tok_buf.at[pl.ds(b * _P, _P)], gsem)

    def gwait(r, tok_buf, gsem):
        for b in range(_B):
            pltpu.make_async_copy(
                tok_hbm.at[idx_all.at[pl.ds(b * _SPW + r * _P, _P)]],
                tok_buf.at[pl.ds(b * _P, _P)], gsem).wait()

    def outs(r, tok_buf, osem):
        for b in range(_B):
            pltpu.async_copy(tok_buf.at[pl.ds(b * _P, _P)],
                             out_hbm.at[b, pl.ds(s_base + r * _P, _P)], osem)

    def owait(r, tok_buf, osem):
        for b in range(_B):
            pltpu.make_async_copy(
                tok_buf.at[pl.ds(b * _P, _P)],
                out_hbm.at[b, pl.ds(s_base + r * _P, _P)], osem).wait()

    def compute_chunk(tok_buf):
        def p1(i):
            zero = jnp.zeros((_L,), jnp.float32)
            a1 = [[zero] * 4 for _ in range(_B)]
            a2 = [[zero] * 4 for _ in range(_B)]
            for cc in range(nchunk):
                sl = pl.ds(cc * _L, _L)
                pv = pos_v[i, sl]
                k = cc & 3
                for b in range(_B):
                    v = tok_buf[b * _P + i, sl] + pv
                    tok_buf[b * _P + i, sl] = v
                    a1[b][k] = a1[b][k] + v
                    a2[b][k] = a2[b][k] + v * v
            for b in range(_B):
                s1 = _tree_sum(a1[b])
                s2 = _tree_sum(a2[b])
                mv = jnp.full((_L,), jnp.sum(s1)) * inv
                qv = jnp.full((_L,), jnp.sum(s2)) * inv
                rr = _nr_rsqrt(qv - mv * mv + jnp.float32(_EPS))
                st_v[b * _P + i, pl.ds(0, _L)] = rr
                st_v[b * _P + i, pl.ds(_L, _L)] = -(mv * rr)

        plsc.parallel_loop(0, _P, 1)(p1)

        strip = 16
        for s in range(nchunk // strip):
            gs = [gam_v[pl.ds((s * strip + j) * _L, _L)] for j in range(strip)]
            bs = [bet_v[pl.ds((s * strip + j) * _L, _L)] for j in range(strip)]

            def p2(rw, _gs=gs, _bs=bs, _s=s):
                rv = st_v[rw, pl.ds(0, _L)]
                nv = st_v[rw, pl.ds(_L, _L)]
                for j in range(strip):
                    sl = pl.ds((_s * strip + j) * _L, _L)
                    x = tok_buf[rw, sl]
                    tok_buf[rw, sl] = (x * rv + nv) * _gs[j] + _bs[j]

            plsc.parallel_loop(0, _KR, 1, unroll=2)(p2)

    def do_round(r, tok_cur, gsem_cur, osem_cur, tok_nxt, gsem_nxt, osem_nxt):
        pltpu.sync_copy(pos_hbm.at[pl.ds(s_base + r * _P, _P)], pos_v)

        gwait(r, tok_cur, gsem_cur)

        @pl.when(r < _NROUND - 1)
        def _():
            @pl.when(r >= 1)
            def _():
                owait(r, tok_nxt, osem_nxt)
            gathers(r + 1, tok_nxt, gsem_nxt)

        compute_chunk(tok_cur)
        outs(r, tok_cur, osem_cur)

    gathers(jnp.int32(0), tok_a, gsem_a)

    def pair(jj, _):
        r0 = jj * 2
        do_round(r0, tok_a, gsem_a, osem_a, tok_b, gsem_b, osem_b)
        do_round(r0 + 1, tok_b, gsem_b, osem_b, tok_a, gsem_a, osem_a)
        return 0

    lax.fori_loop(0, _NROUND // 2, pair, 0)

    owait(jnp.int32(_NROUND - 2), tok_a, osem_a)
    owait(jnp.int32(_NROUND - 1), tok_b, osem_b)


_mesh = plsc.VectorSubcoreMesh(
    core_axis_name="c", subcore_axis_name="s", num_cores=_NC, num_subcores=_NS)

_embed_ln = pl.kernel(
    _body,
    out_type=jax.ShapeDtypeStruct((_B, _S, _H), jnp.float32),
    mesh=_mesh,
    scratch_types=[
        pltpu.VMEM((_B * _SPW,), jnp.int32),
        pltpu.VMEM((_P, _H), jnp.float32),
        pltpu.VMEM((_KR, _H), jnp.float32),
        pltpu.VMEM((_KR, _H), jnp.float32),
        pltpu.VMEM((_H,), jnp.float32),
        pltpu.VMEM((_H,), jnp.float32),
        pltpu.VMEM((_KR, 2 * _L), jnp.float32),
        pltpu.SemaphoreType.DMA,
        pltpu.SemaphoreType.DMA,
        pltpu.SemaphoreType.DMA,
        pltpu.SemaphoreType.DMA,
    ],
    compiler_params=pltpu.CompilerParams(
        use_tc_tiling_on_sc=True, needs_layout_passes=False),
)


def kernel(input_ids, tok_table, pos_table, gamma, beta):
    return _embed_ln(input_ids.astype(jnp.int32), tok_table, pos_table,
                     gamma, beta)


# speedup vs baseline: 1.7607x; 1.6507x over previous
import jax
import jax.numpy as jnp
from jax import lax
from jax.experimental import pallas as pl
from jax.experimental.pallas import tpu as pltpu
from jax.experimental.pallas import tpu_sc as plsc

_B = 4
_S = 8192
_H = 768
_EPS = 1e-12
_NC = 2
_NS = 16
_NW = _NC * _NS
_SPW = _S // _NW
_K = 32
_NPC = _SPW // _K
_L = 16
_NROUND = _NPC * _B


def _nr_rsqrt(v):
    i = lax.bitcast_convert_type(v, jnp.int32)
    y = lax.bitcast_convert_type(
        jnp.int32(0x5F3759DF) - lax.shift_right_arithmetic(i, 1), jnp.float32)
    for _ in range(2):
        y = y * (1.5 - 0.5 * v * y * y)
    return y


def _tree_sum(vs):
    vs = list(vs)
    while len(vs) > 1:
        nxt = [vs[i] + vs[i + 1] for i in range(0, len(vs) - 1, 2)]
        if len(vs) % 2:
            nxt.append(vs[-1])
        vs = nxt
    return vs[0]


def _body(ids_hbm, tok_hbm, pos_hbm, gamma_hbm, beta_hbm, out_hbm,
          idx_all, pos_v, tok_a, tok_b, gam_v, bet_v, r_st, nmr_st,
          gsem_a, gsem_b, osem_a, osem_b):
    cid = lax.axis_index("c")
    sid = lax.axis_index("s")
    wid = sid * _NC + cid
    s_base = wid * _SPW

    pltpu.sync_copy(gamma_hbm, gam_v)
    pltpu.sync_copy(beta_hbm, bet_v)
    for b in range(_B):
        pltpu.sync_copy(ids_hbm.at[b, pl.ds(s_base, _SPW)],
                        idx_all.at[pl.ds(b * _SPW, _SPW)])

    def idx_slice(r):
        pc = lax.shift_right_logical(r, 2)
        b = lax.bitwise_and(r, 3)
        return idx_all.at[pl.ds(b * _SPW + pc * _K, _K)]

    inv = jnp.float32(1.0 / _H)
    nchunk = _H // _L

    def compute_chunk(tok_buf):
        def p1(r):
            vs = []
            for cc in range(nchunk):
                sl = pl.ds(cc * _L, _L)
                v = tok_buf[r, sl] + pos_v[r, sl]
                tok_buf[r, sl] = v
                vs.append(v)
            s1 = _tree_sum(vs)
            s2 = _tree_sum([v * v for v in vs])
            mv = jnp.full((_L,), jnp.sum(s1)) * inv
            qv = jnp.full((_L,), jnp.sum(s2)) * inv
            rr = _nr_rsqrt(qv - mv * mv + jnp.float32(_EPS))
            r_st[r, :] = rr
            nmr_st[r, :] = -(mv * rr)

        plsc.parallel_loop(0, _K, 1, unroll=2)(p1)

        strip = 16
        for s in range(nchunk // strip):
            gs = [gam_v[pl.ds((s * strip + j) * _L, _L)] for j in range(strip)]
            bs = [bet_v[pl.ds((s * strip + j) * _L, _L)] for j in range(strip)]

            def p2(r, _gs=gs, _bs=bs, _s=s):
                rv = r_st[r, :]
                nv = nmr_st[r, :]
                for j in range(strip):
                    sl = pl.ds((_s * strip + j) * _L, _L)
                    x = tok_buf[r, sl]
                    tok_buf[r, sl] = (x * rv + nv) * _gs[j] + _bs[j]

            plsc.parallel_loop(0, _K, 1, unroll=2)(p2)

    def do_round(r, tok_cur, gsem_cur, osem_cur, tok_nxt, gsem_nxt, osem_nxt):
        pc = lax.shift_right_logical(r, 2)
        b = lax.bitwise_and(r, 3)
        s0 = s_base + pc * _K
        out_sl = out_hbm.at[b, pl.ds(s0, _K)]

        @pl.when(b == 0)
        def _():
            pltpu.sync_copy(pos_hbm.at[pl.ds(s0, _K)], pos_v)

        pltpu.make_async_copy(tok_hbm.at[idx_slice(r)], tok_cur, gsem_cur).wait()

        @pl.when(r < _NROUND - 1)
        def _():
            @pl.when(r >= 1)
            def _():
                pltpu.make_async_copy(tok_nxt, out_sl, osem_nxt).wait()
            pltpu.async_copy(tok_hbm.at[idx_slice(r + 1)], tok_nxt, gsem_nxt)

        compute_chunk(tok_cur)
        pltpu.async_copy(tok_cur, out_sl, osem_cur)

    pltpu.async_copy(tok_hbm.at[idx_slice(jnp.int32(0))], tok_a, gsem_a)

    def pair(jj, _):
        r0 = jj * 2
        do_round(r0, tok_a, gsem_a, osem_a, tok_b, gsem_b, osem_b)
        do_round(r0 + 1, tok_b, gsem_b, osem_b, tok_a, gsem_a, osem_a)
        return 0

    lax.fori_loop(0, _NROUND // 2, pair, 0)

    pltpu.make_async_copy(tok_a, out_hbm.at[0, pl.ds(s_base, _K)], osem_a).wait()
    pltpu.make_async_copy(tok_b, out_hbm.at[0, pl.ds(s_base, _K)], osem_b).wait()


_mesh = plsc.VectorSubcoreMesh(
    core_axis_name="c", subcore_axis_name="s", num_cores=_NC, num_subcores=_NS)

_embed_ln = pl.kernel(
    _body,
    out_type=jax.ShapeDtypeStruct((_B, _S, _H), jnp.float32),
    mesh=_mesh,
    scratch_types=[
        pltpu.VMEM((_B * _SPW,), jnp.int32),
        pltpu.VMEM((_K, _H), jnp.float32),
        pltpu.VMEM((_K, _H), jnp.float32),
        pltpu.VMEM((_K, _H), jnp.float32),
        pltpu.VMEM((_H,), jnp.float32),
        pltpu.VMEM((_H,), jnp.float32),
        pltpu.VMEM((_K, _L), jnp.float32),
        pltpu.VMEM((_K, _L), jnp.float32),
        pltpu.SemaphoreType.DMA,
        pltpu.SemaphoreType.DMA,
        pltpu.SemaphoreType.DMA,
        pltpu.SemaphoreType.DMA,
    ],
    compiler_params=pltpu.CompilerParams(
        use_tc_tiling_on_sc=True, needs_layout_passes=False),
)


def kernel(input_ids, tok_table, pos_table, gamma, beta):
    return _embed_ln(input_ids.astype(jnp.int32), tok_table, pos_table,
                     gamma, beta)


# speedup vs baseline: 1.7727x; 1.0068x over previous
import jax
import jax.numpy as jnp
from jax import lax
from jax.experimental import pallas as pl
from jax.experimental.pallas import tpu as pltpu
from jax.experimental.pallas import tpu_sc as plsc

_B = 4
_S = 8192
_H = 768
_EPS = 1e-12
_NC = 2
_NS = 16
_NW = _NC * _NS
_SPW = _S // _NW
_K = 32
_NPC = _SPW // _K
_L = 16
_NROUND = _NPC * _B


def _nr_rsqrt(v):
    i = lax.bitcast_convert_type(v, jnp.int32)
    y = lax.bitcast_convert_type(
        jnp.int32(0x5F3759DF) - lax.shift_right_arithmetic(i, 1), jnp.float32)
    for _ in range(2):
        y = y * (1.5 - 0.5 * v * y * y)
    return y


def _tree_sum(vs):
    vs = list(vs)
    while len(vs) > 1:
        nxt = [vs[i] + vs[i + 1] for i in range(0, len(vs) - 1, 2)]
        if len(vs) % 2:
            nxt.append(vs[-1])
        vs = nxt
    return vs[0]


def _body(ids_hbm, tok_hbm, pos_hbm, gamma_hbm, beta_hbm, out_hbm,
          idx_all, pos_v, tok_a, tok_b, gam_v, bet_v, r_st, nmr_st,
          gsem_a, gsem_b, osem_a, osem_b, psem):
    cid = lax.axis_index("c")
    sid = lax.axis_index("s")
    wid = sid * _NC + cid
    s_base = wid * _SPW

    pltpu.sync_copy(gamma_hbm, gam_v)
    pltpu.sync_copy(beta_hbm, bet_v)
    for b in range(_B):
        pltpu.sync_copy(ids_hbm.at[b, pl.ds(s_base, _SPW)],
                        idx_all.at[pl.ds(b * _SPW, _SPW)])

    def idx_slice(r):
        pc = lax.shift_right_logical(r, 2)
        b = lax.bitwise_and(r, 3)
        return idx_all.at[pl.ds(b * _SPW + pc * _K, _K)]

    inv = jnp.float32(1.0 / _H)
    nchunk = _H // _L

    def compute_chunk(tok_buf):
        def p1(r):
            vs = []
            for cc in range(nchunk):
                sl = pl.ds(cc * _L, _L)
                v = tok_buf[r, sl] + pos_v[r, sl]
                tok_buf[r, sl] = v
                vs.append(v)
            s1 = _tree_sum(vs)
            s2 = _tree_sum([v * v for v in vs])
            mv = jnp.full((_L,), jnp.sum(s1)) * inv
            qv = jnp.full((_L,), jnp.sum(s2)) * inv
            rr = _nr_rsqrt(qv - mv * mv + jnp.float32(_EPS))
            r_st[r, :] = rr
            nmr_st[r, :] = -(mv * rr)

        plsc.parallel_loop(0, _K, 1, unroll=2)(p1)

        strip = 16
        for s in range(nchunk // strip):
            gs = [gam_v[pl.ds((s * strip + j) * _L, _L)] for j in range(strip)]
            bs = [bet_v[pl.ds((s * strip + j) * _L, _L)] for j in range(strip)]

            def p2(r, _gs=gs, _bs=bs, _s=s):
                rv = r_st[r, :]
                nv = nmr_st[r, :]
                for j in range(strip):
                    sl = pl.ds((_s * strip + j) * _L, _L)
                    x = tok_buf[r, sl]
                    tok_buf[r, sl] = (x * rv + nv) * _gs[j] + _bs[j]

            plsc.parallel_loop(0, _K, 1, unroll=2)(p2)

    def do_round(r, tok_cur, gsem_cur, osem_cur, tok_nxt, gsem_nxt, osem_nxt):
        pc = lax.shift_right_logical(r, 2)
        b = lax.bitwise_and(r, 3)
        s0 = s_base + pc * _K
        out_sl = out_hbm.at[b, pl.ds(s0, _K)]

        @pl.when(b == 0)
        def _():
            pltpu.make_async_copy(pos_hbm.at[pl.ds(s0, _K)], pos_v, psem).wait()

        pltpu.make_async_copy(tok_hbm.at[idx_slice(r)], tok_cur, gsem_cur).wait()

        @pl.when(r < _NROUND - 1)
        def _():
            @pl.when(r >= 1)
            def _():
                pltpu.make_async_copy(tok_nxt, out_sl, osem_nxt).wait()
            pltpu.async_copy(tok_hbm.at[idx_slice(r + 1)], tok_nxt, gsem_nxt)

        compute_chunk(tok_cur)
        pltpu.async_copy(tok_cur, out_sl, osem_cur)

        @pl.when(jnp.logical_and(b == 3, r < _NROUND - 1))
        def _():
            pltpu.async_copy(pos_hbm.at[pl.ds(s0 + _K, _K)], pos_v, psem)

    pltpu.async_copy(tok_hbm.at[idx_slice(jnp.int32(0))], tok_a, gsem_a)
    pltpu.async_copy(pos_hbm.at[pl.ds(s_base, _K)], pos_v, psem)

    def pair(jj, _):
        r0 = jj * 2
        do_round(r0, tok_a, gsem_a, osem_a, tok_b, gsem_b, osem_b)
        do_round(r0 + 1, tok_b, gsem_b, osem_b, tok_a, gsem_a, osem_a)
        return 0

    lax.fori_loop(0, _NROUND // 2, pair, 0)

    pltpu.make_async_copy(tok_a, out_hbm.at[0, pl.ds(s_base, _K)], osem_a).wait()
    pltpu.make_async_copy(tok_b, out_hbm.at[0, pl.ds(s_base, _K)], osem_b).wait()


_mesh = plsc.VectorSubcoreMesh(
    core_axis_name="c", subcore_axis_name="s", num_cores=_NC, num_subcores=_NS)

_embed_ln = pl.kernel(
    _body,
    out_type=jax.ShapeDtypeStruct((_B, _S, _H), jnp.float32),
    mesh=_mesh,
    scratch_types=[
        pltpu.VMEM((_B * _SPW,), jnp.int32),
        pltpu.VMEM((_K, _H), jnp.float32),
        pltpu.VMEM((_K, _H), jnp.float32),
        pltpu.VMEM((_K, _H), jnp.float32),
        pltpu.VMEM((_H,), jnp.float32),
        pltpu.VMEM((_H,), jnp.float32),
        pltpu.VMEM((_K, _L), jnp.float32),
        pltpu.VMEM((_K, _L), jnp.float32),
        pltpu.SemaphoreType.DMA,
        pltpu.SemaphoreType.DMA,
        pltpu.SemaphoreType.DMA,
        pltpu.SemaphoreType.DMA,
        pltpu.SemaphoreType.DMA,
    ],
    compiler_params=pltpu.CompilerParams(
        use_tc_tiling_on_sc=True, needs_layout_passes=False),
)


def kernel(input_ids, tok_table, pos_table, gamma, beta):
    return _embed_ln(input_ids.astype(jnp.int32), tok_table, pos_table,
                     gamma, beta)
